# Initial kernel scaffold; baseline (speedup 1.0000x reference)
import jax
import jax.numpy as jnp
from jax.experimental import pallas as pl


def kernel(input, weight):
    raise NotImplementedError("write your pallas kernel here")


# speedup vs baseline: 4.7160x; 4.7160x over previous
import functools

import jax
import jax.numpy as jnp
from jax import lax
from jax.experimental import pallas as pl
from jax.experimental.pallas import tpu as pltpu
from jax.experimental.pallas import tpu_sc as plsc

_CHUNK = 128


@functools.lru_cache(maxsize=None)
def _build_gather(d: int, b: int):
    info = plsc.get_sparse_core_info()
    nc, ns = info.num_cores, info.num_subcores
    nw = nc * ns
    assert b % (nw * _CHUNK) == 0
    b_per_w = b // nw
    n_chunks = b_per_w // _CHUNK
    mesh = plsc.VectorSubcoreMesh(core_axis_name="c", subcore_axis_name="s")

    @functools.partial(
        pl.kernel,
        mesh=mesh,
        out_type=jax.ShapeDtypeStruct((b, d), jnp.float32),
        scratch_types=[
            pltpu.VMEM((_CHUNK,), jnp.int32),
            pltpu.VMEM((_CHUNK, d), jnp.float32),
            pltpu.SemaphoreType.DMA,
        ],
    )
    def gather_k(table_hbm, idx_hbm, out_hbm, idx_v, rows_v, sem):
        wid = lax.axis_index("s") * nc + lax.axis_index("c")
        w_base = wid * b_per_w

        def body(i, carry):
            base = w_base + i * _CHUNK
            pltpu.sync_copy(idx_hbm.at[pl.ds(base, _CHUNK)], idx_v)
            pltpu.async_copy(table_hbm.at[idx_v], rows_v, sem).wait()
            pltpu.sync_copy(rows_v, out_hbm.at[pl.ds(base, _CHUNK)])
            return carry

        lax.fori_loop(0, n_chunks, body, 0)

    return gather_k


def kernel(input, weight):
    bsz, hist = input.shape
    _, d = weight.shape
    b = bsz * hist
    idx_flat = input.reshape(b)
    out = _build_gather(d, b)(weight, idx_flat)
    return out.reshape(bsz, hist, d)


# speedup vs baseline: 5.3055x; 1.1250x over previous
import functools

import jax
import jax.numpy as jnp
from jax import lax
from jax.experimental import pallas as pl
from jax.experimental.pallas import tpu as pltpu
from jax.experimental.pallas import tpu_sc as plsc

_CHUNK = 128


@functools.lru_cache(maxsize=None)
def _build_gather(d: int, b: int):
    info = plsc.get_sparse_core_info()
    nc, ns = info.num_cores, info.num_subcores
    nw = nc * ns
    assert b % (nw * 2 * _CHUNK) == 0
    b_per_w = b // nw
    n_chunks = b_per_w // _CHUNK
    n_pairs = n_chunks // 2
    mesh = plsc.VectorSubcoreMesh(core_axis_name="c", subcore_axis_name="s")

    @functools.partial(
        pl.kernel,
        mesh=mesh,
        out_type=jax.ShapeDtypeStruct((b, d), jnp.float32),
        scratch_types=[
            pltpu.VMEM((n_chunks, _CHUNK), jnp.int32),
            pltpu.VMEM((_CHUNK, d), jnp.float32),
            pltpu.VMEM((_CHUNK, d), jnp.float32),
            pltpu.SemaphoreType.DMA,
            pltpu.SemaphoreType.DMA,
            pltpu.SemaphoreType.DMA,
            pltpu.SemaphoreType.DMA,
        ],
    )
    def gather_k(table_hbm, idx_hbm, out_hbm, idx_v, rows0, rows1,
                 sg0, sg1, ss0, ss1):
        wid = lax.axis_index("s") * nc + lax.axis_index("c")
        crow0 = wid * n_chunks
        base0 = wid * b_per_w

        pltpu.sync_copy(idx_hbm.at[pl.ds(crow0, n_chunks)], idx_v)

        def g_start(ci, rows_b, sg):
            pltpu.async_copy(table_hbm.at[idx_v.at[ci]], rows_b, sg)

        def g_wait(ci, rows_b, sg):
            pltpu.make_async_copy(table_hbm.at[idx_v.at[ci]], rows_b, sg).wait()

        def s_start(ci, rows_b, ss):
            pltpu.async_copy(
                rows_b, out_hbm.at[pl.ds(base0 + ci * _CHUNK, _CHUNK)], ss)

        def s_wait(ci, rows_b, ss):
            pltpu.make_async_copy(
                rows_b, out_hbm.at[pl.ds(base0 + ci * _CHUNK, _CHUNK)], ss).wait()

        g_start(0, rows0, sg0)
        g_wait(0, rows0, sg0)
        g_start(1, rows1, sg1)
        s_start(0, rows0, ss0)

        def body(j, carry):
            c1 = 2 * j + 1
            c2 = c1 + 1
            c3 = c1 + 2
            g_wait(c1, rows1, sg1)
            s_wait(c1 - 1, rows0, ss0)
            g_start(c2, rows0, sg0)
            s_start(c1, rows1, ss1)
            g_wait(c2, rows0, sg0)
            s_wait(c1, rows1, ss1)
            g_start(c3, rows1, sg1)
            s_start(c2, rows0, ss0)
            return carry

        lax.fori_loop(0, n_pairs - 1, body, 0)

        g_wait(n_chunks - 1, rows1, sg1)
        s_start(n_chunks - 1, rows1, ss1)
        s_wait(n_chunks - 2, rows0, ss0)
        s_wait(n_chunks - 1, rows1, ss1)

    return gather_k


def kernel(input, weight):
    bsz, hist = input.shape
    _, d = weight.shape
    b = bsz * hist
    idx2d = input.reshape(b // _CHUNK, _CHUNK)
    out = _build_gather(d, b)(weight, idx2d)
    return out.reshape(bsz, hist, d)


# speedup vs baseline: 5.3524x; 1.0088x over previous
import functools

import jax
import jax.numpy as jnp
from jax import lax
from jax.experimental import pallas as pl
from jax.experimental.pallas import tpu as pltpu
from jax.experimental.pallas import tpu_sc as plsc

_CHUNK = 128
_K = 2


@functools.lru_cache(maxsize=None)
def _build_gather(d: int, b: int):
    info = plsc.get_sparse_core_info()
    nc, ns = info.num_cores, info.num_subcores
    nw = nc * ns
    sc_rows = _K * _CHUNK
    assert b % (nw * 2 * sc_rows) == 0
    b_per_w = b // nw
    n_chunks = b_per_w // _CHUNK
    n_super = n_chunks // _K
    n_pairs = n_super // 2
    mesh = plsc.VectorSubcoreMesh(core_axis_name="c", subcore_axis_name="s")

    @functools.partial(
        pl.kernel,
        mesh=mesh,
        out_type=jax.ShapeDtypeStruct((b, d), jnp.float32),
        scratch_types=[
            pltpu.VMEM((b_per_w,), jnp.int32),
            pltpu.VMEM((sc_rows, d), jnp.float32),
            pltpu.VMEM((sc_rows, d), jnp.float32),
            pltpu.SemaphoreType.DMA,
            pltpu.SemaphoreType.DMA,
            pltpu.SemaphoreType.DMA,
            pltpu.SemaphoreType.DMA,
        ],
    )
    def gather_k(table_hbm, idx_hbm, out_hbm, idx_v, rows0, rows1,
                 sg0, sg1, ss0, ss1):
        wid = lax.axis_index("s") * nc + lax.axis_index("c")
        base0 = wid * b_per_w

        pltpu.sync_copy(idx_hbm.at[pl.ds(base0, b_per_w)], idx_v)

        def g_start(si, rows_b, sg):
            pltpu.async_copy(
                table_hbm.at[idx_v.at[pl.ds(si * sc_rows, sc_rows)]], rows_b, sg)

        def g_wait(si, rows_b, sg):
            pltpu.make_async_copy(
                table_hbm.at[idx_v.at[pl.ds(si * sc_rows, sc_rows)]], rows_b, sg).wait()

        def s_start(si, rows_b, ss):
            pltpu.async_copy(
                rows_b, out_hbm.at[pl.ds(base0 + si * sc_rows, sc_rows)], ss)

        def s_wait(si, rows_b, ss):
            pltpu.make_async_copy(
                rows_b, out_hbm.at[pl.ds(base0 + si * sc_rows, sc_rows)], ss).wait()

        g_start(0, rows0, sg0)
        g_wait(0, rows0, sg0)
        g_start(1, rows1, sg1)
        s_start(0, rows0, ss0)

        def body(j, carry):
            c1 = 2 * j + 1
            c2 = c1 + 1
            c3 = c1 + 2
            g_wait(c1, rows1, sg1)
            s_wait(c1 - 1, rows0, ss0)
            g_start(c2, rows0, sg0)
            s_start(c1, rows1, ss1)
            g_wait(c2, rows0, sg0)
            s_wait(c1, rows1, ss1)
            g_start(c3, rows1, sg1)
            s_start(c2, rows0, ss0)
            return carry

        lax.fori_loop(0, n_pairs - 1, body, 0)

        g_wait(n_super - 1, rows1, sg1)
        s_start(n_super - 1, rows1, ss1)
        s_wait(n_super - 2, rows0, ss0)
        s_wait(n_super - 1, rows1, ss1)

    return gather_k


def kernel(input, weight):
    bsz, hist = input.shape
    _, d = weight.shape
    b = bsz * hist
    idx_flat = input.reshape(b)
    out = _build_gather(d, b)(weight, idx_flat)
    return out.reshape(bsz, hist, d)


# speedup vs baseline: 15.4698x; 2.8903x over previous
import functools

import jax
import jax.numpy as jnp
from jax import lax
from jax.experimental import pallas as pl
from jax.experimental.pallas import tpu as pltpu
from jax.experimental.pallas import tpu_sc as plsc

_CHUNK = 128
_K = 2


@functools.lru_cache(maxsize=None)
def _build_gather(v: int, d: int, b: int):
    info = plsc.get_sparse_core_info()
    nc, ns = info.num_cores, info.num_subcores
    nw = nc * ns
    sc_rows = _K * _CHUNK
    assert b % (nw * 2 * sc_rows) == 0
    b_per_w = b // nw
    n_chunks = b_per_w // _CHUNK
    n_super = n_chunks // _K
    n_pairs = n_super // 2
    mesh = plsc.VectorSubcoreMesh(core_axis_name="c", subcore_axis_name="s")

    @functools.partial(
        pl.kernel,
        mesh=mesh,
        out_type=jax.ShapeDtypeStruct((b, d), jnp.float32),
        scratch_types=[
            pltpu.VMEM((b_per_w,), jnp.int32),
            pltpu.VMEM((sc_rows, d), jnp.float32),
            pltpu.VMEM((sc_rows, d), jnp.float32),
            pltpu.VMEM_SHARED((v, d), jnp.float32),
            pltpu.SemaphoreType.DMA,
            pltpu.SemaphoreType.DMA,
            pltpu.SemaphoreType.DMA,
            pltpu.SemaphoreType.DMA,
        ],
    )
    def gather_k(table_hbm, idx_hbm, out_hbm, idx_v, rows0, rows1, tab_sp,
                 sg0, sg1, ss0, ss1):
        wid = lax.axis_index("s") * nc + lax.axis_index("c")
        base0 = wid * b_per_w

        @pl.when(lax.axis_index("s") == 0)
        def _():
            pltpu.sync_copy(table_hbm, tab_sp)

        pltpu.sync_copy(idx_hbm.at[pl.ds(base0, b_per_w)], idx_v)
        plsc.subcore_barrier()

        def g_start(si, rows_b, sg):
            pltpu.async_copy(
                tab_sp.at[idx_v.at[pl.ds(si * sc_rows, sc_rows)]], rows_b, sg)

        def g_wait(si, rows_b, sg):
            pltpu.make_async_copy(
                tab_sp.at[idx_v.at[pl.ds(si * sc_rows, sc_rows)]], rows_b, sg).wait()

        def s_start(si, rows_b, ss):
            pltpu.async_copy(
                rows_b, out_hbm.at[pl.ds(base0 + si * sc_rows, sc_rows)], ss)

        def s_wait(si, rows_b, ss):
            pltpu.make_async_copy(
                rows_b, out_hbm.at[pl.ds(base0 + si * sc_rows, sc_rows)], ss).wait()

        g_start(0, rows0, sg0)
        g_wait(0, rows0, sg0)
        g_start(1, rows1, sg1)
        s_start(0, rows0, ss0)

        def body(j, carry):
            c1 = 2 * j + 1
            c2 = c1 + 1
            c3 = c1 + 2
            g_wait(c1, rows1, sg1)
            s_wait(c1 - 1, rows0, ss0)
            g_start(c2, rows0, sg0)
            s_start(c1, rows1, ss1)
            g_wait(c2, rows0, sg0)
            s_wait(c1, rows1, ss1)
            g_start(c3, rows1, sg1)
            s_start(c2, rows0, ss0)
            return carry

        lax.fori_loop(0, n_pairs - 1, body, 0)

        g_wait(n_super - 1, rows1, sg1)
        s_start(n_super - 1, rows1, ss1)
        s_wait(n_super - 2, rows0, ss0)
        s_wait(n_super - 1, rows1, ss1)

    return gather_k


def kernel(input, weight):
    bsz, hist = input.shape
    _, d = weight.shape
    b = bsz * hist
    idx_flat = input.reshape(b)
    out = _build_gather(weight.shape[0], d, b)(weight, idx_flat)
    return out.reshape(bsz, hist, d)
